# Initial kernel scaffold; baseline (speedup 1.0000x reference)
import jax
import jax.numpy as jnp
from jax.experimental import pallas as pl


def kernel(atom_indices, embedding_table):
    raise NotImplementedError("write your pallas kernel here")


# speedup vs baseline: 1.2431x; 1.2431x over previous
import functools

import jax
import jax.numpy as jnp
from jax import lax
from jax.experimental import pallas as pl
from jax.experimental.pallas import tpu as pltpu
from jax.experimental.pallas import tpu_sc as plsc

N = 100000
V = 119
D = 128
NC = 2
NS = 16
NW = NC * NS
CHUNK = 112
NCHUNK = 28
BPW = CHUNK * NCHUNK
B_PAD = BPW * NW

_mesh = plsc.VectorSubcoreMesh(core_axis_name="c", subcore_axis_name="s")


@functools.partial(
    pl.kernel,
    mesh=_mesh,
    out_type=jax.ShapeDtypeStruct((B_PAD, D), jnp.float32),
    scratch_types=[
        pltpu.VMEM((CHUNK,), jnp.int32),
        pltpu.VMEM((CHUNK, D), jnp.float32),
        pltpu.SemaphoreType.DMA,
    ],
)
def _sc_gather(idx_hbm, table_hbm, out_hbm, idx_v, rows_v, sem):
    wid = lax.axis_index("s") * NC + lax.axis_index("c")
    base = wid * BPW

    def body(k, carry):
        off = base + k * CHUNK
        pltpu.sync_copy(idx_hbm.at[pl.ds(off, CHUNK)], idx_v)
        pltpu.async_copy(table_hbm.at[idx_v], rows_v, sem).wait()
        pltpu.sync_copy(rows_v, out_hbm.at[pl.ds(off, CHUNK)])
        return carry

    lax.fori_loop(0, NCHUNK, body, 0)


def kernel(atom_indices, embedding_table):
    idx = jnp.pad(atom_indices.astype(jnp.int32), (0, B_PAD - N))
    out = _sc_gather(idx, embedding_table)
    return out[:N, :, None]


# speedup vs baseline: 1.2895x; 1.0373x over previous
import functools

import jax
import jax.numpy as jnp
from jax import lax
from jax.experimental import pallas as pl
from jax.experimental.pallas import tpu as pltpu
from jax.experimental.pallas import tpu_sc as plsc

N = 100000
D = 128
NC = 2
NS = 16
NW = NC * NS
CHUNK = 112
NCHUNK = 28
NBUF = 4
NGEN = NCHUNK // NBUF
BPW = CHUNK * NCHUNK
B_PAD = BPW * NW

_mesh = plsc.VectorSubcoreMesh(core_axis_name="c", subcore_axis_name="s")


@functools.partial(
    pl.kernel,
    mesh=_mesh,
    out_type=jax.ShapeDtypeStruct((B_PAD, D), jnp.float32),
    scratch_types=(
        [pltpu.VMEM((BPW,), jnp.int32)]
        + [pltpu.VMEM((CHUNK, D), jnp.float32) for _ in range(NBUF)]
        + [pltpu.SemaphoreType.DMA for _ in range(2 * NBUF)]
    ),
)
def _sc_gather(idx_hbm, table_hbm, out_hbm, idx_v, *bufs_and_sems):
    rows = bufs_and_sems[:NBUF]
    gsem = bufs_and_sems[NBUF:2 * NBUF]
    osem = bufs_and_sems[2 * NBUF:]
    wid = lax.axis_index("s") * NC + lax.axis_index("c")
    base = wid * BPW

    pltpu.sync_copy(idx_hbm.at[pl.ds(base, BPW)], idx_v)

    def gather(chunk, b):
        pltpu.async_copy(
            table_hbm.at[idx_v.at[pl.ds(chunk * CHUNK, CHUNK)]],
            rows[b], gsem[b])

    def scatter(chunk, b):
        pltpu.async_copy(
            rows[b], out_hbm.at[pl.ds(base + chunk * CHUNK, CHUNK)], osem[b])

    for b in range(NBUF):
        gather(b, b)

    def body(t, carry):
        g0 = t * NBUF
        for b in range(NBUF):
            pltpu.make_async_copy(
                table_hbm.at[idx_v.at[pl.ds(0, CHUNK)]], rows[b],
                gsem[b]).wait()
            scatter(g0 + b, b)
        for b in range(NBUF):
            pltpu.make_async_copy(
                rows[b], out_hbm.at[pl.ds(0, CHUNK)], osem[b]).wait()
            gather(g0 + NBUF + b, b)
        return carry

    lax.fori_loop(0, NGEN - 1, body, 0)

    g0 = (NGEN - 1) * NBUF
    for b in range(NBUF):
        pltpu.make_async_copy(
            table_hbm.at[idx_v.at[pl.ds(0, CHUNK)]], rows[b], gsem[b]).wait()
        scatter(g0 + b, b)
    for b in range(NBUF):
        pltpu.make_async_copy(
            rows[b], out_hbm.at[pl.ds(0, CHUNK)], osem[b]).wait()


def kernel(atom_indices, embedding_table):
    idx = jnp.pad(atom_indices.astype(jnp.int32), (0, B_PAD - N))
    out = _sc_gather(idx, embedding_table)
    return out[:N, :, None]


# speedup vs baseline: 3.2254x; 2.5013x over previous
import functools

import jax
import jax.numpy as jnp
from jax import lax
from jax.experimental import pallas as pl
from jax.experimental.pallas import tpu as pltpu
from jax.experimental.pallas import tpu_sc as plsc

N = 100000
D = 128
NC = 2
NS = 16
NW = NC * NS
CHUNK = 112
NCHUNK = 28
NBUF = 4
NGEN = NCHUNK // NBUF
BPW = CHUNK * NCHUNK
B_PAD = BPW * NW

_mesh = plsc.VectorSubcoreMesh(core_axis_name="c", subcore_axis_name="s")


@functools.partial(
    pl.kernel,
    mesh=_mesh,
    out_type=jax.ShapeDtypeStruct((B_PAD, D), jnp.float32),
    scratch_types=(
        [pltpu.VMEM((BPW,), jnp.int32),
         pltpu.VMEM_SHARED((119, D), jnp.float32)]
        + [pltpu.VMEM((CHUNK, D), jnp.float32) for _ in range(NBUF)]
        + [pltpu.SemaphoreType.DMA for _ in range(2 * NBUF)]
    ),
)
def _sc_gather(idx_hbm, table_hbm, out_hbm, idx_v, table_v, *bufs_and_sems):
    rows = bufs_and_sems[:NBUF]
    gsem = bufs_and_sems[NBUF:2 * NBUF]
    osem = bufs_and_sems[2 * NBUF:]
    wid = lax.axis_index("s") * NC + lax.axis_index("c")
    base = wid * BPW

    pltpu.sync_copy(idx_hbm.at[pl.ds(base, BPW)], idx_v)

    @pl.when(lax.axis_index("s") == 0)
    def _stage_table():
        pltpu.sync_copy(table_hbm, table_v)

    plsc.subcore_barrier()

    def gather(chunk, b):
        pltpu.async_copy(
            table_v.at[idx_v.at[pl.ds(chunk * CHUNK, CHUNK)]],
            rows[b], gsem[b])

    def scatter(chunk, b):
        pltpu.async_copy(
            rows[b], out_hbm.at[pl.ds(base + chunk * CHUNK, CHUNK)], osem[b])

    for b in range(NBUF):
        gather(b, b)

    def body(t, carry):
        g0 = t * NBUF
        for b in range(NBUF):
            pltpu.make_async_copy(
                table_v.at[idx_v.at[pl.ds(0, CHUNK)]], rows[b],
                gsem[b]).wait()
            scatter(g0 + b, b)
        for b in range(NBUF):
            pltpu.make_async_copy(
                rows[b], out_hbm.at[pl.ds(0, CHUNK)], osem[b]).wait()
            gather(g0 + NBUF + b, b)
        return carry

    lax.fori_loop(0, NGEN - 1, body, 0)

    g0 = (NGEN - 1) * NBUF
    for b in range(NBUF):
        pltpu.make_async_copy(
            table_v.at[idx_v.at[pl.ds(0, CHUNK)]], rows[b], gsem[b]).wait()
        scatter(g0 + b, b)
    for b in range(NBUF):
        pltpu.make_async_copy(
            rows[b], out_hbm.at[pl.ds(0, CHUNK)], osem[b]).wait()


def kernel(atom_indices, embedding_table):
    idx = jnp.pad(atom_indices.astype(jnp.int32), (0, B_PAD - N))
    out = _sc_gather(idx, embedding_table)
    return out[:N, :, None]


# speedup vs baseline: 5.6885x; 1.7637x over previous
import functools

import jax
import jax.numpy as jnp
from jax import lax
from jax.experimental import pallas as pl
from jax.experimental.pallas import tpu as pltpu
from jax.experimental.pallas import tpu_sc as plsc

N = 100000
D = 128
NC = 2
NS = 16
NW = NC * NS
CHUNK = 112
NCHUNK = 28
NBUF = 4
NGEN = NCHUNK // NBUF
BPW = CHUNK * NCHUNK
TAIL_IDX = N - (NW - 1) * BPW
TAILC = N % CHUNK
assert TAIL_IDX % 8 == 0 and TAILC % 8 == 0

_mesh = plsc.VectorSubcoreMesh(core_axis_name="c", subcore_axis_name="s")


@functools.partial(
    pl.kernel,
    mesh=_mesh,
    out_type=jax.ShapeDtypeStruct((N, D), jnp.float32),
    scratch_types=(
        [pltpu.VMEM((BPW,), jnp.int32),
         pltpu.VMEM_SHARED((119, D), jnp.float32)]
        + [pltpu.VMEM((CHUNK, D), jnp.float32) for _ in range(NBUF)]
        + [pltpu.SemaphoreType.DMA for _ in range(2 * NBUF)]
    ),
)
def _sc_gather(idx_hbm, table_hbm, out_hbm, idx_v, table_v, *bufs_and_sems):
    rows = bufs_and_sems[:NBUF]
    gsem = bufs_and_sems[NBUF:2 * NBUF]
    osem = bufs_and_sems[2 * NBUF:]
    wid = lax.axis_index("s") * NC + lax.axis_index("c")
    base = wid * BPW
    is_last = wid == NW - 1

    @pl.when(jnp.logical_not(is_last))
    def _stage_idx():
        pltpu.sync_copy(idx_hbm.at[pl.ds(base, BPW)], idx_v)

    @pl.when(is_last)
    def _stage_idx_tail():
        pltpu.sync_copy(idx_hbm.at[pl.ds(base, TAIL_IDX)],
                        idx_v.at[pl.ds(0, TAIL_IDX)])

    @pl.when(lax.axis_index("s") == 0)
    def _stage_table():
        pltpu.sync_copy(table_hbm, table_v)

    plsc.subcore_barrier()

    def gather(chunk, b, size=CHUNK):
        pltpu.async_copy(
            table_v.at[idx_v.at[pl.ds(chunk * CHUNK, size)]],
            rows[b].at[pl.ds(0, size)], gsem[b])

    def scatter(chunk, b, size=CHUNK):
        pltpu.async_copy(
            rows[b].at[pl.ds(0, size)],
            out_hbm.at[pl.ds(base + chunk * CHUNK, size)], osem[b])

    def wait_g(b, size=CHUNK):
        pltpu.make_async_copy(
            table_v.at[idx_v.at[pl.ds(0, size)]],
            rows[b].at[pl.ds(0, size)], gsem[b]).wait()

    def wait_o(b, size=CHUNK):
        pltpu.make_async_copy(
            rows[b].at[pl.ds(0, size)],
            out_hbm.at[pl.ds(0, size)], osem[b]).wait()

    for b in range(NBUF):
        gather(b, b)

    def body(t, carry):
        g0 = t * NBUF
        for b in range(NBUF):
            wait_g(b)
            scatter(g0 + b, b)
        for b in range(NBUF):
            wait_o(b)
            nxt = g0 + NBUF + b
            off = base + nxt * CHUNK
            @pl.when(off + CHUNK <= N)
            def _full(nxt=nxt, b=b):
                gather(nxt, b)

            @pl.when(jnp.logical_and(off < N, off + CHUNK > N))
            def _tail(nxt=nxt, b=b):
                gather(nxt, b, TAILC)

        return carry

    lax.fori_loop(0, NGEN - 1, body, 0)

    g0 = (NGEN - 1) * NBUF
    for b in range(NBUF):
        off = base + (g0 + b) * CHUNK
        full = off + CHUNK <= N
        part = jnp.logical_and(off < N, off + CHUNK > N)

        @pl.when(full)
        def _full(b=b, c=g0 + b):
            wait_g(b)
            scatter(c, b)

        @pl.when(part)
        def _part(b=b, c=g0 + b):
            wait_g(b, TAILC)
            scatter(c, b, TAILC)

    for b in range(NBUF):
        off = base + (g0 + b) * CHUNK
        full = off + CHUNK <= N
        part = jnp.logical_and(off < N, off + CHUNK > N)

        @pl.when(full)
        def _full(b=b):
            wait_o(b)

        @pl.when(part)
        def _part(b=b):
            wait_o(b, TAILC)


def kernel(atom_indices, embedding_table):
    out = _sc_gather(atom_indices.astype(jnp.int32), embedding_table)
    return out[..., None]


# speedup vs baseline: 5.7118x; 1.0041x over previous
import functools

import jax
import jax.numpy as jnp
from jax import lax
from jax.experimental import pallas as pl
from jax.experimental.pallas import tpu as pltpu
from jax.experimental.pallas import tpu_sc as plsc

N = 100000
D = 128
NC = 2
NS = 16
NW = NC * NS
CHUNK = 112
NCHUNK = 28
NBUF = 7
NGEN = NCHUNK // NBUF
BPW = CHUNK * NCHUNK
TAIL_IDX = N - (NW - 1) * BPW
TAILC = N % CHUNK
assert TAIL_IDX % 8 == 0 and TAILC % 8 == 0

_mesh = plsc.VectorSubcoreMesh(core_axis_name="c", subcore_axis_name="s")


@functools.partial(
    pl.kernel,
    mesh=_mesh,
    out_type=jax.ShapeDtypeStruct((N, D), jnp.float32),
    scratch_types=(
        [pltpu.VMEM((BPW,), jnp.int32),
         pltpu.VMEM_SHARED((119, D), jnp.float32)]
        + [pltpu.VMEM((CHUNK, D), jnp.float32) for _ in range(NBUF)]
        + [pltpu.SemaphoreType.DMA for _ in range(2 * NBUF)]
    ),
)
def _sc_gather(idx_hbm, table_hbm, out_hbm, idx_v, table_v, *bufs_and_sems):
    rows = bufs_and_sems[:NBUF]
    gsem = bufs_and_sems[NBUF:2 * NBUF]
    osem = bufs_and_sems[2 * NBUF:]
    wid = lax.axis_index("s") * NC + lax.axis_index("c")
    base = wid * BPW
    is_last = wid == NW - 1

    @pl.when(jnp.logical_not(is_last))
    def _stage_idx():
        pltpu.sync_copy(idx_hbm.at[pl.ds(base, BPW)], idx_v)

    @pl.when(is_last)
    def _stage_idx_tail():
        pltpu.sync_copy(idx_hbm.at[pl.ds(base, TAIL_IDX)],
                        idx_v.at[pl.ds(0, TAIL_IDX)])

    @pl.when(lax.axis_index("s") == 0)
    def _stage_table():
        pltpu.sync_copy(table_hbm, table_v)

    plsc.subcore_barrier()

    def gather(chunk, b, size=CHUNK):
        pltpu.async_copy(
            table_v.at[idx_v.at[pl.ds(chunk * CHUNK, size)]],
            rows[b].at[pl.ds(0, size)], gsem[b])

    def scatter(chunk, b, size=CHUNK):
        pltpu.async_copy(
            rows[b].at[pl.ds(0, size)],
            out_hbm.at[pl.ds(base + chunk * CHUNK, size)], osem[b])

    def wait_g(b, size=CHUNK):
        pltpu.make_async_copy(
            table_v.at[idx_v.at[pl.ds(0, size)]],
            rows[b].at[pl.ds(0, size)], gsem[b]).wait()

    def wait_o(b, size=CHUNK):
        pltpu.make_async_copy(
            rows[b].at[pl.ds(0, size)],
            out_hbm.at[pl.ds(0, size)], osem[b]).wait()

    for b in range(NBUF):
        gather(b, b)

    def body(t, carry):
        g0 = t * NBUF
        for b in range(NBUF):
            wait_g(b)
            scatter(g0 + b, b)
        for b in range(NBUF):
            wait_o(b)
            nxt = g0 + NBUF + b
            off = base + nxt * CHUNK
            @pl.when(off + CHUNK <= N)
            def _full(nxt=nxt, b=b):
                gather(nxt, b)

            @pl.when(jnp.logical_and(off < N, off + CHUNK > N))
            def _tail(nxt=nxt, b=b):
                gather(nxt, b, TAILC)

        return carry

    lax.fori_loop(0, NGEN - 1, body, 0)

    g0 = (NGEN - 1) * NBUF
    for b in range(NBUF):
        off = base + (g0 + b) * CHUNK
        full = off + CHUNK <= N
        part = jnp.logical_and(off < N, off + CHUNK > N)

        @pl.when(full)
        def _full(b=b, c=g0 + b):
            wait_g(b)
            scatter(c, b)

        @pl.when(part)
        def _part(b=b, c=g0 + b):
            wait_g(b, TAILC)
            scatter(c, b, TAILC)

    for b in range(NBUF):
        off = base + (g0 + b) * CHUNK
        full = off + CHUNK <= N
        part = jnp.logical_and(off < N, off + CHUNK > N)

        @pl.when(full)
        def _full(b=b):
            wait_o(b)

        @pl.when(part)
        def _part(b=b):
            wait_o(b, TAILC)


def kernel(atom_indices, embedding_table):
    out = _sc_gather(atom_indices.astype(jnp.int32), embedding_table)
    return out[..., None]


# speedup vs baseline: 5.8089x; 1.0170x over previous
import functools

import jax
import jax.numpy as jnp
from jax import lax
from jax.experimental import pallas as pl
from jax.experimental.pallas import tpu as pltpu
from jax.experimental.pallas import tpu_sc as plsc

N = 100000
D = 128
NC = 2
NS = 16
NW = NC * NS
CHUNK = 112
NCHUNK = 28
NBUF = 7
NGEN = NCHUNK // NBUF
BPW = CHUNK * NCHUNK
TAIL_IDX = N - (NW - 1) * BPW
TAILC = N % CHUNK
assert TAIL_IDX % 8 == 0 and TAILC % 8 == 0

_mesh = plsc.VectorSubcoreMesh(core_axis_name="c", subcore_axis_name="s")


@functools.partial(
    pl.kernel,
    mesh=_mesh,
    out_type=jax.ShapeDtypeStruct((N, D), jnp.float32),
    scratch_types=(
        [pltpu.VMEM((BPW,), jnp.int32),
         pltpu.VMEM_SHARED((119, D), jnp.float32)]
        + [pltpu.VMEM((CHUNK, D), jnp.float32) for _ in range(NBUF)]
        + [pltpu.SemaphoreType.DMA for _ in range(2 * NBUF)]
    ),
)
def _sc_gather(idx_hbm, table_hbm, out_hbm, idx_v, table_v, *bufs_and_sems):
    rows = bufs_and_sems[:NBUF]
    gsem = bufs_and_sems[NBUF:2 * NBUF]
    osem = bufs_and_sems[2 * NBUF:]
    wid = lax.axis_index("s") * NC + lax.axis_index("c")
    base = wid * BPW
    is_last = wid == NW - 1

    @pl.when(lax.axis_index("s") == 0)
    def _stage_table():
        pltpu.async_copy(table_hbm, table_v, bufs_and_sems[NBUF])

    @pl.when(jnp.logical_not(is_last))
    def _stage_idx():
        pltpu.sync_copy(idx_hbm.at[pl.ds(base, BPW)], idx_v)

    @pl.when(is_last)
    def _stage_idx_tail():
        pltpu.sync_copy(idx_hbm.at[pl.ds(base, TAIL_IDX)],
                        idx_v.at[pl.ds(0, TAIL_IDX)])

    @pl.when(lax.axis_index("s") == 0)
    def _stage_table_wait():
        pltpu.make_async_copy(table_hbm, table_v, bufs_and_sems[NBUF]).wait()

    plsc.subcore_barrier()

    def gather(chunk, b, size=CHUNK):
        pltpu.async_copy(
            table_v.at[idx_v.at[pl.ds(chunk * CHUNK, size)]],
            rows[b].at[pl.ds(0, size)], gsem[b])

    def scatter(chunk, b, size=CHUNK):
        pltpu.async_copy(
            rows[b].at[pl.ds(0, size)],
            out_hbm.at[pl.ds(base + chunk * CHUNK, size)], osem[b])

    def wait_g(b, size=CHUNK):
        pltpu.make_async_copy(
            table_v.at[idx_v.at[pl.ds(0, size)]],
            rows[b].at[pl.ds(0, size)], gsem[b]).wait()

    def wait_o(b, size=CHUNK):
        pltpu.make_async_copy(
            rows[b].at[pl.ds(0, size)],
            out_hbm.at[pl.ds(0, size)], osem[b]).wait()

    for b in range(NBUF):
        gather(b, b)

    def body(t, carry):
        g0 = t * NBUF
        for b in range(NBUF):
            wait_g(b)
            scatter(g0 + b, b)
        for b in range(NBUF):
            wait_o(b)
            nxt = g0 + NBUF + b
            off = base + nxt * CHUNK
            @pl.when(off + CHUNK <= N)
            def _full(nxt=nxt, b=b):
                gather(nxt, b)

            @pl.when(jnp.logical_and(off < N, off + CHUNK > N))
            def _tail(nxt=nxt, b=b):
                gather(nxt, b, TAILC)

        return carry

    lax.fori_loop(0, NGEN - 1, body, 0)

    g0 = (NGEN - 1) * NBUF
    for b in range(NBUF):
        off = base + (g0 + b) * CHUNK
        full = off + CHUNK <= N
        part = jnp.logical_and(off < N, off + CHUNK > N)

        @pl.when(full)
        def _full(b=b, c=g0 + b):
            wait_g(b)
            scatter(c, b)

        @pl.when(part)
        def _part(b=b, c=g0 + b):
            wait_g(b, TAILC)
            scatter(c, b, TAILC)

    for b in range(NBUF):
        off = base + (g0 + b) * CHUNK
        full = off + CHUNK <= N
        part = jnp.logical_and(off < N, off + CHUNK > N)

        @pl.when(full)
        def _full(b=b):
            wait_o(b)

        @pl.when(part)
        def _part(b=b):
            wait_o(b, TAILC)


def kernel(atom_indices, embedding_table):
    out = _sc_gather(atom_indices.astype(jnp.int32), embedding_table)
    return out[..., None]
